# baseline (speedup 1.0000x reference)
import functools

import jax
import jax.numpy as jnp
from jax import lax
from jax.experimental import pallas as pl
from jax.experimental.pallas import tpu as pltpu
from jax.experimental.pallas import tpu_sc as plsc

NUM_CORES = 2
NUM_SUBCORES = 16
NW = NUM_CORES * NUM_SUBCORES

BLK = 128
L = 16


def _emb_body(seq, d, ids_hbm, table_hbm, out_hbm,
              idx_v, idx2_v, rows0, rows1, t0, t1,
              gsem0, gsem1, osem0, osem1):
  wid = lax.axis_index("s") * NUM_CORES + lax.axis_index("c")

  pltpu.sync_copy(ids_hbm.at[:, pl.ds(wid * BLK, BLK)], idx_v)

  lanes = lax.iota(jnp.int32, L)
  rows_j = [lanes + (j * L) for j in range(BLK // L)]

  def prep_idx(s, slot):
    for j in range(BLK // L):
      v = idx_v[s, pl.ds(j * L, L)]
      idx2_v[slot, 0, pl.ds(j * L, L)] = lax.bitwise_or(
          lax.bitwise_and(lax.shift_right_logical(v, 1), -256),
          lax.bitwise_and(v, 255))
      idx2_v[slot, 1, pl.ds(j * L, L)] = lax.bitwise_and(
          lax.shift_right_logical(v, 2), 64)

  def fire_gather(slot, rows_v, gsem):
    return pltpu.async_copy(
        table_hbm.at[idx2_v.at[slot, 0]], rows_v, gsem)

  def transpose_into(rows_v, t_v, slot):
    and64 = [idx2_v[slot, 1, pl.ds(j * L, L)] for j in range(BLK // L)]

    def dbody(dd, carry):
      dmod = lax.bitwise_and(lanes + dd, d - 1)
      for j in range(BLK // L):
        vals = plsc.load_gather(rows_v, [rows_j[j], and64[j] + dmod])
        plsc.store_scatter(t_v, [dmod, rows_j[j]], vals)
      return carry
    lax.fori_loop(0, d, dbody, 0)

  def fire_out(s, t_v, osem):
    return pltpu.async_copy(
        t_v, out_hbm.at[s, :, pl.ds(wid * BLK, BLK)], osem)

  def wait_out(s, t_v, osem):
    pltpu.make_async_copy(
        t_v, out_hbm.at[s, :, pl.ds(wid * BLK, BLK)], osem).wait()

  def drain_gather(slot, rows_v, gsem):
    pltpu.make_async_copy(
        table_hbm.at[idx2_v.at[slot, 0]], rows_v, gsem).wait()

  prep_idx(0, 0)
  fire_gather(0, rows0, gsem0)
  prep_idx(1, 1)
  fire_gather(1, rows1, gsem1)

  drain_gather(0, rows0, gsem0)
  transpose_into(rows0, t0, 0)
  fire_out(0, t0, osem0)
  prep_idx(2, 0)
  fire_gather(0, rows0, gsem0)
  drain_gather(1, rows1, gsem1)
  transpose_into(rows1, t1, 1)
  fire_out(1, t1, osem1)
  prep_idx(3, 1)
  fire_gather(1, rows1, gsem1)

  def pair_body(i, carry):
    s = 2 * i
    drain_gather(0, rows0, gsem0)
    wait_out(s - 2, t0, osem0)
    transpose_into(rows0, t0, 0)
    fire_out(s, t0, osem0)
    prep_idx(lax.min(s + 2, seq - 1), 0)
    fire_gather(0, rows0, gsem0)
    drain_gather(1, rows1, gsem1)
    wait_out(s - 1, t1, osem1)
    transpose_into(rows1, t1, 1)
    fire_out(s + 1, t1, osem1)
    prep_idx(lax.min(s + 3, seq - 1), 1)
    fire_gather(1, rows1, gsem1)
    return carry

  lax.fori_loop(1, seq // 2, pair_body, 0)

  drain_gather(0, rows0, gsem0)
  drain_gather(1, rows1, gsem1)
  wait_out(seq - 2, t0, osem0)
  wait_out(seq - 1, t1, osem1)


def _prep_body(tn_ref, out_ref):
  x = tn_ref[...]
  eye = jnp.eye(64, dtype=jnp.float32)
  xt = lax.dot_general(x, eye, (((0,), (0,)), ((), ())),
                       preferred_element_type=jnp.float32)
  out_ref[...] = jnp.concatenate([xt[:256], xt[256:]], axis=1)


def _prep_table(table):
  n_rows, d = table.shape
  cb = 512
  n_blk = (n_rows + cb - 1) // cb
  tn = jnp.transpose(table)
  return pl.pallas_call(
      _prep_body,
      grid=(n_blk,),
      in_specs=[pl.BlockSpec((d, cb), lambda i: (0, i))],
      out_specs=pl.BlockSpec((cb // 2, 2 * d), lambda i: (i, 0)),
      out_shape=jax.ShapeDtypeStruct((n_blk * (cb // 2), 2 * d),
                                     jnp.float32),
  )(tn)


@jax.jit
def kernel(input_ids, table):
  batch, seq = input_ids.shape
  n_rows, d = table.shape
  assert batch % (NW * BLK) == 0 or batch == NW * BLK

  ids_t = jnp.transpose(input_ids)
  tbl2 = _prep_table(table)

  mesh = plsc.VectorSubcoreMesh(core_axis_name="c", subcore_axis_name="s")
  run = pl.kernel(
      functools.partial(_emb_body, seq, d),
      out_type=jax.ShapeDtypeStruct((seq, d, batch), jnp.float32),
      mesh=mesh,
      compiler_params=pltpu.CompilerParams(use_tc_tiling_on_sc=True,
                                           needs_layout_passes=False),
      scratch_types=[
          pltpu.VMEM((seq, BLK), jnp.int32),
          pltpu.VMEM((2, 2, BLK), jnp.int32),
          pltpu.VMEM((BLK, 2 * d), jnp.float32),
          pltpu.VMEM((BLK, 2 * d), jnp.float32),
          pltpu.VMEM((d, BLK), jnp.float32),
          pltpu.VMEM((d, BLK), jnp.float32),
          pltpu.SemaphoreType.DMA,
          pltpu.SemaphoreType.DMA,
          pltpu.SemaphoreType.DMA,
          pltpu.SemaphoreType.DMA,
      ],
  )
  out_t = run(ids_t, tbl2)
  return jnp.transpose(out_t, (2, 0, 1))


# speedup vs baseline: 2.2190x; 2.2190x over previous
import functools

import jax
import jax.numpy as jnp
from jax import lax
from jax.experimental import pallas as pl
from jax.experimental.pallas import tpu as pltpu
from jax.experimental.pallas import tpu_sc as plsc

NUM_CORES = 2
NUM_SUBCORES = 16
NW = NUM_CORES * NUM_SUBCORES

BLK = 128
L = 16


def _emb_body(seq, d, ids_hbm, table_hbm, out_hbm,
              idx_v, idx2_v, rows0, rows1, t0, t1,
              gsem0, gsem1, osem0, osem1):
  wid = lax.axis_index("s") * NUM_CORES + lax.axis_index("c")

  pltpu.sync_copy(ids_hbm.at[:, pl.ds(wid * BLK, BLK)], idx_v)

  lanes = lax.iota(jnp.int32, L)
  rows_j = [lanes + (j * L) for j in range(BLK // L)]

  hb = PREP_CB // 2

  def prep_idx(s, slot):
    for j in range(BLK // L):
      v = idx_v[s, pl.ds(j * L, L)]
      idx2_v[slot, 0, pl.ds(j * L, L)] = lax.bitwise_or(
          lax.bitwise_and(lax.shift_right_logical(v, 1), -hb),
          lax.bitwise_and(v, hb - 1))
      idx2_v[slot, 1, pl.ds(j * L, L)] = lax.bitwise_and(
          lax.shift_right_logical(v, 5), 64)

  def fire_gather(slot, rows_v, gsem):
    return pltpu.async_copy(
        table_hbm.at[idx2_v.at[slot, 0]], rows_v, gsem)

  def transpose_into(rows_v, t_v, slot):
    and64 = [idx2_v[slot, 1, pl.ds(j * L, L)] for j in range(BLK // L)]

    def dbody(dd, carry):
      dmod = lax.bitwise_and(lanes + dd, d - 1)
      for j in range(BLK // L):
        vals = plsc.load_gather(rows_v, [rows_j[j], and64[j] + dmod])
        plsc.store_scatter(t_v, [dmod, rows_j[j]], vals)
      return carry
    lax.fori_loop(0, d, dbody, 0)

  def fire_out(s, t_v, osem):
    return pltpu.async_copy(
        t_v, out_hbm.at[s, :, pl.ds(wid * BLK, BLK)], osem)

  def wait_out(s, t_v, osem):
    pltpu.make_async_copy(
        t_v, out_hbm.at[s, :, pl.ds(wid * BLK, BLK)], osem).wait()

  def drain_gather(slot, rows_v, gsem):
    pltpu.make_async_copy(
        table_hbm.at[idx2_v.at[slot, 0]], rows_v, gsem).wait()

  prep_idx(0, 0)
  fire_gather(0, rows0, gsem0)
  prep_idx(1, 1)
  fire_gather(1, rows1, gsem1)

  drain_gather(0, rows0, gsem0)
  transpose_into(rows0, t0, 0)
  fire_out(0, t0, osem0)
  prep_idx(2, 0)
  fire_gather(0, rows0, gsem0)
  drain_gather(1, rows1, gsem1)
  transpose_into(rows1, t1, 1)
  fire_out(1, t1, osem1)
  prep_idx(3, 1)
  fire_gather(1, rows1, gsem1)

  def pair_body(i, carry):
    s = 2 * i
    drain_gather(0, rows0, gsem0)
    wait_out(s - 2, t0, osem0)
    transpose_into(rows0, t0, 0)
    fire_out(s, t0, osem0)
    prep_idx(lax.min(s + 2, seq - 1), 0)
    fire_gather(0, rows0, gsem0)
    drain_gather(1, rows1, gsem1)
    wait_out(s - 1, t1, osem1)
    transpose_into(rows1, t1, 1)
    fire_out(s + 1, t1, osem1)
    prep_idx(lax.min(s + 3, seq - 1), 1)
    fire_gather(1, rows1, gsem1)
    return carry

  lax.fori_loop(1, seq // 2, pair_body, 0)

  drain_gather(0, rows0, gsem0)
  drain_gather(1, rows1, gsem1)
  wait_out(seq - 2, t0, osem0)
  wait_out(seq - 1, t1, osem1)


PREP_CB = 4096


def _prep_body(tn_ref, out_ref):
  hb = PREP_CB // 2
  xt = jnp.transpose(tn_ref[...])
  out_ref[...] = jnp.concatenate([xt[:hb], xt[hb:]], axis=1)


def _prep_table(table):
  n_rows, d = table.shape
  cb = PREP_CB
  n_blk = (n_rows + cb - 1) // cb
  tn = jnp.transpose(table)
  return pl.pallas_call(
      _prep_body,
      grid=(n_blk,),
      in_specs=[pl.BlockSpec((d, cb), lambda i: (0, i))],
      out_specs=pl.BlockSpec((cb // 2, 2 * d), lambda i: (i, 0)),
      out_shape=jax.ShapeDtypeStruct((n_blk * (cb // 2), 2 * d),
                                     jnp.float32),
  )(tn)


@jax.jit
def kernel(input_ids, table):
  batch, seq = input_ids.shape
  n_rows, d = table.shape
  assert batch % (NW * BLK) == 0 or batch == NW * BLK

  ids_t = jnp.transpose(input_ids)
  tbl2 = _prep_table(table)

  mesh = plsc.VectorSubcoreMesh(core_axis_name="c", subcore_axis_name="s")
  run = pl.kernel(
      functools.partial(_emb_body, seq, d),
      out_type=jax.ShapeDtypeStruct((seq, d, batch), jnp.float32),
      mesh=mesh,
      compiler_params=pltpu.CompilerParams(use_tc_tiling_on_sc=True,
                                           needs_layout_passes=False),
      scratch_types=[
          pltpu.VMEM((seq, BLK), jnp.int32),
          pltpu.VMEM((2, 2, BLK), jnp.int32),
          pltpu.VMEM((BLK, 2 * d), jnp.float32),
          pltpu.VMEM((BLK, 2 * d), jnp.float32),
          pltpu.VMEM((d, BLK), jnp.float32),
          pltpu.VMEM((d, BLK), jnp.float32),
          pltpu.SemaphoreType.DMA,
          pltpu.SemaphoreType.DMA,
          pltpu.SemaphoreType.DMA,
          pltpu.SemaphoreType.DMA,
      ],
  )
  out_t = run(ids_t, tbl2)
  return jnp.transpose(out_t, (2, 0, 1))


# speedup vs baseline: 2.4391x; 1.0992x over previous
import functools

import jax
import jax.numpy as jnp
from jax import lax
from jax.experimental import pallas as pl
from jax.experimental.pallas import tpu as pltpu
from jax.experimental.pallas import tpu_sc as plsc

NUM_CORES = 2
NUM_SUBCORES = 16
NW = NUM_CORES * NUM_SUBCORES

BLK = 128
L = 16


def _emb_body(seq, d, ids_hbm, table_hbm, out_hbm,
              idx_v, idx2_v, rows0, rows1, t0, t1,
              gsem0, gsem1, osem0, osem1):
  wid = lax.axis_index("s") * NUM_CORES + lax.axis_index("c")

  pltpu.sync_copy(ids_hbm.at[:, pl.ds(wid * BLK, BLK)], idx_v)

  lanes = lax.iota(jnp.int32, L)
  rows_j = [lanes + (j * L) for j in range(BLK // L)]

  hb = PREP_CB // 2

  def prep_idx(s, slot):
    for j in range(BLK // L):
      v = idx_v[s, pl.ds(j * L, L)]
      idx2_v[slot, 0, pl.ds(j * L, L)] = lax.bitwise_or(
          lax.bitwise_and(lax.shift_right_logical(v, 1), -hb),
          lax.bitwise_and(v, hb - 1))
      idx2_v[slot, 1, pl.ds(j * L, L)] = lax.bitwise_and(
          lax.shift_right_logical(v, hb.bit_length() - 7), 64)

  def fire_gather(slot, rows_v, gsem):
    return pltpu.async_copy(
        table_hbm.at[idx2_v.at[slot, 0]], rows_v, gsem)

  def transpose_into(rows_v, t_v, slot):
    and64 = [idx2_v[slot, 1, pl.ds(j * L, L)] for j in range(BLK // L)]

    def dbody(dd, carry):
      dmod = lax.bitwise_and(lanes + dd, d - 1)
      for j in range(BLK // L):
        vals = plsc.load_gather(rows_v, [rows_j[j], and64[j] + dmod])
        plsc.store_scatter(t_v, [dmod, rows_j[j]], vals)
      return carry
    lax.fori_loop(0, d, dbody, 0)

  def fire_out(s, t_v, osem):
    return pltpu.async_copy(
        t_v, out_hbm.at[s, :, pl.ds(wid * BLK, BLK)], osem)

  def wait_out(s, t_v, osem):
    pltpu.make_async_copy(
        t_v, out_hbm.at[s, :, pl.ds(wid * BLK, BLK)], osem).wait()

  def drain_gather(slot, rows_v, gsem):
    pltpu.make_async_copy(
        table_hbm.at[idx2_v.at[slot, 0]], rows_v, gsem).wait()

  prep_idx(0, 0)
  fire_gather(0, rows0, gsem0)
  prep_idx(1, 1)
  fire_gather(1, rows1, gsem1)

  drain_gather(0, rows0, gsem0)
  transpose_into(rows0, t0, 0)
  fire_out(0, t0, osem0)
  prep_idx(2, 0)
  fire_gather(0, rows0, gsem0)
  drain_gather(1, rows1, gsem1)
  transpose_into(rows1, t1, 1)
  fire_out(1, t1, osem1)
  prep_idx(3, 1)
  fire_gather(1, rows1, gsem1)

  def pair_body(i, carry):
    s = 2 * i
    drain_gather(0, rows0, gsem0)
    wait_out(s - 2, t0, osem0)
    transpose_into(rows0, t0, 0)
    fire_out(s, t0, osem0)
    prep_idx(lax.min(s + 2, seq - 1), 0)
    fire_gather(0, rows0, gsem0)
    drain_gather(1, rows1, gsem1)
    wait_out(s - 1, t1, osem1)
    transpose_into(rows1, t1, 1)
    fire_out(s + 1, t1, osem1)
    prep_idx(lax.min(s + 3, seq - 1), 1)
    fire_gather(1, rows1, gsem1)
    return carry

  lax.fori_loop(1, seq // 2, pair_body, 0)

  drain_gather(0, rows0, gsem0)
  drain_gather(1, rows1, gsem1)
  wait_out(seq - 2, t0, osem0)
  wait_out(seq - 1, t1, osem1)


PREP_CB = 8192


def _prep_body(tn_ref, out_ref):
  hb = PREP_CB // 2
  xt = jnp.transpose(tn_ref[...])
  out_ref[...] = jnp.concatenate([xt[:hb], xt[hb:]], axis=1)


def _prep_table(table):
  n_rows, d = table.shape
  cb = PREP_CB
  n_blk = (n_rows + cb - 1) // cb
  tn = jnp.transpose(table)
  return pl.pallas_call(
      _prep_body,
      grid=(n_blk,),
      in_specs=[pl.BlockSpec((d, cb), lambda i: (0, i))],
      out_specs=pl.BlockSpec((cb // 2, 2 * d), lambda i: (i, 0)),
      out_shape=jax.ShapeDtypeStruct((n_blk * (cb // 2), 2 * d),
                                     jnp.float32),
  )(tn)


@jax.jit
def kernel(input_ids, table):
  batch, seq = input_ids.shape
  n_rows, d = table.shape
  assert batch % (NW * BLK) == 0 or batch == NW * BLK

  ids_t = jnp.transpose(input_ids)
  tbl2 = _prep_table(table)

  mesh = plsc.VectorSubcoreMesh(core_axis_name="c", subcore_axis_name="s")
  run = pl.kernel(
      functools.partial(_emb_body, seq, d),
      out_type=jax.ShapeDtypeStruct((seq, d, batch), jnp.float32),
      mesh=mesh,
      compiler_params=pltpu.CompilerParams(use_tc_tiling_on_sc=True,
                                           needs_layout_passes=False),
      scratch_types=[
          pltpu.VMEM((seq, BLK), jnp.int32),
          pltpu.VMEM((2, 2, BLK), jnp.int32),
          pltpu.VMEM((BLK, 2 * d), jnp.float32),
          pltpu.VMEM((BLK, 2 * d), jnp.float32),
          pltpu.VMEM((d, BLK), jnp.float32),
          pltpu.VMEM((d, BLK), jnp.float32),
          pltpu.SemaphoreType.DMA,
          pltpu.SemaphoreType.DMA,
          pltpu.SemaphoreType.DMA,
          pltpu.SemaphoreType.DMA,
      ],
  )
  out_t = run(ids_t, tbl2)
  return jnp.transpose(out_t, (2, 0, 1))


# speedup vs baseline: 2.5621x; 1.0504x over previous
import functools

import jax
import jax.numpy as jnp
from jax import lax
from jax.experimental import pallas as pl
from jax.experimental.pallas import tpu as pltpu
from jax.experimental.pallas import tpu_sc as plsc

NUM_CORES = 2
NUM_SUBCORES = 16
NW = NUM_CORES * NUM_SUBCORES

BLK = 128
L = 16


def _emb_body(seq, d, ids_hbm, table_hbm, out_hbm,
              idx_v, idx2_v, rows0, rows1, t0, t1,
              gsem0, gsem1, osem0, osem1):
  wid = lax.axis_index("s") * NUM_CORES + lax.axis_index("c")

  pltpu.sync_copy(ids_hbm.at[:, pl.ds(wid * BLK, BLK)], idx_v)

  lanes = lax.iota(jnp.int32, L)
  rows_j = [lanes + (j * L) for j in range(BLK // L)]

  hb = PREP_CB // 2

  def prep_idx(s, slot):
    for j in range(BLK // L):
      v = idx_v[s, pl.ds(j * L, L)]
      idx2_v[slot, 0, pl.ds(j * L, L)] = lax.bitwise_or(
          lax.bitwise_and(lax.shift_right_logical(v, 1), -hb),
          lax.bitwise_and(v, hb - 1))
      idx2_v[slot, 1, pl.ds(j * L, L)] = lax.bitwise_and(
          lax.shift_right_logical(v, hb.bit_length() - 7), 64)

  def fire_gather(slot, rows_v, gsem):
    return pltpu.async_copy(
        table_hbm.at[idx2_v.at[slot, 0]], rows_v, gsem)

  def transpose_into(rows_v, t_v, slot):
    and64 = [idx2_v[slot, 1, pl.ds(j * L, L)] for j in range(BLK // L)]

    def dbody(dd, carry):
      dmod = lax.bitwise_and(lanes + dd, d - 1)
      for j in range(BLK // L):
        vals = plsc.load_gather(rows_v, [rows_j[j], and64[j] + dmod])
        plsc.store_scatter(t_v, [dmod, rows_j[j]], vals)
      return carry
    lax.fori_loop(0, d, dbody, 0)

  def fire_out(s, t_v, osem):
    return pltpu.async_copy(
        t_v, out_hbm.at[s, :, pl.ds(wid * BLK, BLK)], osem)

  def wait_out(s, t_v, osem):
    pltpu.make_async_copy(
        t_v, out_hbm.at[s, :, pl.ds(wid * BLK, BLK)], osem).wait()

  def drain_gather(slot, rows_v, gsem):
    pltpu.make_async_copy(
        table_hbm.at[idx2_v.at[slot, 0]], rows_v, gsem).wait()

  prep_idx(0, 0)
  fire_gather(0, rows0, gsem0)
  prep_idx(1, 1)
  fire_gather(1, rows1, gsem1)

  drain_gather(0, rows0, gsem0)
  transpose_into(rows0, t0, 0)
  fire_out(0, t0, osem0)
  prep_idx(2, 0)
  fire_gather(0, rows0, gsem0)
  drain_gather(1, rows1, gsem1)
  transpose_into(rows1, t1, 1)
  fire_out(1, t1, osem1)
  prep_idx(3, 1)
  fire_gather(1, rows1, gsem1)

  def pair_body(i, carry):
    s = 2 * i
    drain_gather(0, rows0, gsem0)
    wait_out(s - 2, t0, osem0)
    transpose_into(rows0, t0, 0)
    fire_out(s, t0, osem0)
    prep_idx(lax.min(s + 2, seq - 1), 0)
    fire_gather(0, rows0, gsem0)
    drain_gather(1, rows1, gsem1)
    wait_out(s - 1, t1, osem1)
    transpose_into(rows1, t1, 1)
    fire_out(s + 1, t1, osem1)
    prep_idx(lax.min(s + 3, seq - 1), 1)
    fire_gather(1, rows1, gsem1)
    return carry

  lax.fori_loop(1, seq // 2, pair_body, 0)

  drain_gather(0, rows0, gsem0)
  drain_gather(1, rows1, gsem1)
  wait_out(seq - 2, t0, osem0)
  wait_out(seq - 1, t1, osem1)


PREP_CB = 16384


def _prep_body(tn_ref, out_ref):
  hb = PREP_CB // 2
  xt = jnp.transpose(tn_ref[...])
  out_ref[...] = jnp.concatenate([xt[:hb], xt[hb:]], axis=1)


def _prep_table(table):
  n_rows, d = table.shape
  cb = PREP_CB
  n_blk = (n_rows + cb - 1) // cb
  tn = jnp.transpose(table)
  return pl.pallas_call(
      _prep_body,
      grid=(n_blk,),
      in_specs=[pl.BlockSpec((d, cb), lambda i: (0, i))],
      out_specs=pl.BlockSpec((cb // 2, 2 * d), lambda i: (i, 0)),
      out_shape=jax.ShapeDtypeStruct((n_blk * (cb // 2), 2 * d),
                                     jnp.float32),
  )(tn)


@jax.jit
def kernel(input_ids, table):
  batch, seq = input_ids.shape
  n_rows, d = table.shape
  assert batch % (NW * BLK) == 0 or batch == NW * BLK

  ids_t = jnp.transpose(input_ids)
  tbl2 = _prep_table(table)

  mesh = plsc.VectorSubcoreMesh(core_axis_name="c", subcore_axis_name="s")
  run = pl.kernel(
      functools.partial(_emb_body, seq, d),
      out_type=jax.ShapeDtypeStruct((seq, d, batch), jnp.float32),
      mesh=mesh,
      compiler_params=pltpu.CompilerParams(use_tc_tiling_on_sc=True,
                                           needs_layout_passes=False),
      scratch_types=[
          pltpu.VMEM((seq, BLK), jnp.int32),
          pltpu.VMEM((2, 2, BLK), jnp.int32),
          pltpu.VMEM((BLK, 2 * d), jnp.float32),
          pltpu.VMEM((BLK, 2 * d), jnp.float32),
          pltpu.VMEM((d, BLK), jnp.float32),
          pltpu.VMEM((d, BLK), jnp.float32),
          pltpu.SemaphoreType.DMA,
          pltpu.SemaphoreType.DMA,
          pltpu.SemaphoreType.DMA,
          pltpu.SemaphoreType.DMA,
      ],
  )
  out_t = run(ids_t, tbl2)
  return jnp.transpose(out_t, (2, 0, 1))


# speedup vs baseline: 2.6189x; 1.0221x over previous
import functools

import jax
import jax.numpy as jnp
from jax import lax
from jax.experimental import pallas as pl
from jax.experimental.pallas import tpu as pltpu
from jax.experimental.pallas import tpu_sc as plsc

NUM_CORES = 2
NUM_SUBCORES = 16
NW = NUM_CORES * NUM_SUBCORES

BLK = 128
L = 16


def _emb_body(seq, d, ids_hbm, table_hbm, out_hbm,
              idx_v, idx2_v, rows0, rows1, t0, t1,
              gsem0, gsem1, osem0, osem1):
  wid = lax.axis_index("s") * NUM_CORES + lax.axis_index("c")

  pltpu.sync_copy(ids_hbm.at[:, pl.ds(wid * BLK, BLK)], idx_v)

  lanes = lax.iota(jnp.int32, L)
  rows_j = [lanes + (j * L) for j in range(BLK // L)]

  hb = PREP_CB // 2

  def prep_idx(s, slot):
    for j in range(BLK // L):
      v = idx_v[s, pl.ds(j * L, L)]
      idx2_v[slot, 0, pl.ds(j * L, L)] = lax.bitwise_or(
          lax.bitwise_and(lax.shift_right_logical(v, 1), -hb),
          lax.bitwise_and(v, hb - 1))
      idx2_v[slot, 1, pl.ds(j * L, L)] = lax.bitwise_and(
          lax.shift_right_logical(v, hb.bit_length() - 7), 64)

  def fire_gather(slot, rows_v, gsem):
    return pltpu.async_copy(
        table_hbm.at[idx2_v.at[slot, 0]], rows_v, gsem)

  def transpose_into(rows_v, t_v, slot):
    and64 = [idx2_v[slot, 1, pl.ds(j * L, L)] for j in range(BLK // L)]

    def dbody(dd, carry):
      dmod = lax.bitwise_and(lanes + dd, d - 1)
      for j in range(BLK // L):
        vals = plsc.load_gather(rows_v, [rows_j[j], and64[j] + dmod])
        plsc.store_scatter(t_v, [dmod, rows_j[j]], vals)
      return carry
    lax.fori_loop(0, d, dbody, 0)

  def fire_out(s, t_v, osem):
    return pltpu.async_copy(
        t_v, out_hbm.at[s, :, pl.ds(wid * BLK, BLK)], osem)

  def wait_out(s, t_v, osem):
    pltpu.make_async_copy(
        t_v, out_hbm.at[s, :, pl.ds(wid * BLK, BLK)], osem).wait()

  def drain_gather(slot, rows_v, gsem):
    pltpu.make_async_copy(
        table_hbm.at[idx2_v.at[slot, 0]], rows_v, gsem).wait()

  prep_idx(0, 0)
  fire_gather(0, rows0, gsem0)
  prep_idx(1, 1)
  fire_gather(1, rows1, gsem1)

  drain_gather(0, rows0, gsem0)
  transpose_into(rows0, t0, 0)
  fire_out(0, t0, osem0)
  prep_idx(2, 0)
  fire_gather(0, rows0, gsem0)
  drain_gather(1, rows1, gsem1)
  transpose_into(rows1, t1, 1)
  fire_out(1, t1, osem1)
  prep_idx(3, 1)
  fire_gather(1, rows1, gsem1)

  def pair_body(i, carry):
    s = 2 * i
    drain_gather(0, rows0, gsem0)
    wait_out(s - 2, t0, osem0)
    transpose_into(rows0, t0, 0)
    fire_out(s, t0, osem0)
    prep_idx(lax.min(s + 2, seq - 1), 0)
    fire_gather(0, rows0, gsem0)
    drain_gather(1, rows1, gsem1)
    wait_out(s - 1, t1, osem1)
    transpose_into(rows1, t1, 1)
    fire_out(s + 1, t1, osem1)
    prep_idx(lax.min(s + 3, seq - 1), 1)
    fire_gather(1, rows1, gsem1)
    return carry

  lax.fori_loop(1, seq // 2, pair_body, 0)

  drain_gather(0, rows0, gsem0)
  drain_gather(1, rows1, gsem1)
  wait_out(seq - 2, t0, osem0)
  wait_out(seq - 1, t1, osem1)


PREP_CB = 32768


def _prep_body(tn_ref, out_ref):
  hb = PREP_CB // 2
  xt = jnp.transpose(tn_ref[...])
  out_ref[...] = jnp.concatenate([xt[:hb], xt[hb:]], axis=1)


def _prep_table(table):
  n_rows, d = table.shape
  cb = PREP_CB
  n_blk = (n_rows + cb - 1) // cb
  tn = jnp.transpose(table)
  return pl.pallas_call(
      _prep_body,
      grid=(n_blk,),
      in_specs=[pl.BlockSpec((d, cb), lambda i: (0, i))],
      out_specs=pl.BlockSpec((cb // 2, 2 * d), lambda i: (i, 0)),
      out_shape=jax.ShapeDtypeStruct((n_blk * (cb // 2), 2 * d),
                                     jnp.float32),
  )(tn)


@jax.jit
def kernel(input_ids, table):
  batch, seq = input_ids.shape
  n_rows, d = table.shape
  assert batch % (NW * BLK) == 0 or batch == NW * BLK

  ids_t = jnp.transpose(input_ids)
  tbl2 = _prep_table(table)

  mesh = plsc.VectorSubcoreMesh(core_axis_name="c", subcore_axis_name="s")
  run = pl.kernel(
      functools.partial(_emb_body, seq, d),
      out_type=jax.ShapeDtypeStruct((seq, d, batch), jnp.float32),
      mesh=mesh,
      compiler_params=pltpu.CompilerParams(use_tc_tiling_on_sc=True,
                                           needs_layout_passes=False),
      scratch_types=[
          pltpu.VMEM((seq, BLK), jnp.int32),
          pltpu.VMEM((2, 2, BLK), jnp.int32),
          pltpu.VMEM((BLK, 2 * d), jnp.float32),
          pltpu.VMEM((BLK, 2 * d), jnp.float32),
          pltpu.VMEM((d, BLK), jnp.float32),
          pltpu.VMEM((d, BLK), jnp.float32),
          pltpu.SemaphoreType.DMA,
          pltpu.SemaphoreType.DMA,
          pltpu.SemaphoreType.DMA,
          pltpu.SemaphoreType.DMA,
      ],
  )
  out_t = run(ids_t, tbl2)
  return jnp.transpose(out_t, (2, 0, 1))


# speedup vs baseline: 2.6440x; 1.0096x over previous
import functools

import jax
import jax.numpy as jnp
from jax import lax
from jax.experimental import pallas as pl
from jax.experimental.pallas import tpu as pltpu
from jax.experimental.pallas import tpu_sc as plsc

NUM_CORES = 2
NUM_SUBCORES = 16
NW = NUM_CORES * NUM_SUBCORES

BLK = 128
L = 16


NSLOT = 3


def _emb_body(seq, d, ids_hbm, table_hbm, out_hbm,
              idx_v, idx2_v, rows0, rows1, rows2, t0, t1, t2,
              gsem0, gsem1, gsem2, osem0, osem1, osem2):
  wid = lax.axis_index("s") * NUM_CORES + lax.axis_index("c")
  rows = (rows0, rows1, rows2)
  ts = (t0, t1, t2)
  gsems = (gsem0, gsem1, gsem2)
  osems = (osem0, osem1, osem2)

  pltpu.sync_copy(ids_hbm.at[:, pl.ds(wid * BLK, BLK)], idx_v)

  lanes = lax.iota(jnp.int32, L)
  rows_j = [lanes + (j * L) for j in range(BLK // L)]

  hb = PREP_CB // 2

  def prep_idx(s, slot):
    for j in range(BLK // L):
      v = idx_v[s, pl.ds(j * L, L)]
      idx2_v[slot, 0, pl.ds(j * L, L)] = lax.bitwise_or(
          lax.bitwise_and(lax.shift_right_logical(v, 1), -hb),
          lax.bitwise_and(v, hb - 1))
      idx2_v[slot, 1, pl.ds(j * L, L)] = lax.bitwise_and(
          lax.shift_right_logical(v, hb.bit_length() - 7), 64)

  def fire_gather(slot):
    return pltpu.async_copy(
        table_hbm.at[idx2_v.at[slot, 0]], rows[slot], gsems[slot])

  def drain_gather(slot):
    pltpu.make_async_copy(
        table_hbm.at[idx2_v.at[slot, 0]], rows[slot], gsems[slot]).wait()

  def transpose_into(slot):
    rows_v, t_v = rows[slot], ts[slot]
    and64 = [idx2_v[slot, 1, pl.ds(j * L, L)] for j in range(BLK // L)]

    def dbody(ii, carry):
      for u in range(2):
        dmod = lax.bitwise_and(lanes + (2 * ii + u), d - 1)
        for j in range(BLK // L):
          vals = plsc.load_gather(rows_v, [rows_j[j], and64[j] + dmod])
          plsc.store_scatter(t_v, [dmod, rows_j[j]], vals)
      return carry
    lax.fori_loop(0, d // 2, dbody, 0)

  def fire_out(s, slot):
    return pltpu.async_copy(
        ts[slot], out_hbm.at[s, :, pl.ds(wid * BLK, BLK)], osems[slot])

  def wait_out(s, slot):
    pltpu.make_async_copy(
        ts[slot], out_hbm.at[s, :, pl.ds(wid * BLK, BLK)],
        osems[slot]).wait()

  def process(s, slot, first_round):
    drain_gather(slot)
    if not first_round:
      wait_out(s - NSLOT, slot)
    transpose_into(slot)
    fire_out(s, slot)
    prep_idx(lax.min(s + NSLOT, seq - 1), slot)
    fire_gather(slot)

  for k in range(NSLOT):
    prep_idx(k, k)
    fire_gather(k)
  for k in range(NSLOT):
    process(k, k, first_round=True)

  def tri_body(i, carry):
    s0 = NSLOT * i
    for k in range(NSLOT):
      process(s0 + k, k, first_round=False)
    return carry

  lax.fori_loop(1, (seq - 2) // NSLOT, tri_body, 0)

  for s, slot in ((seq - 2, (seq - 2) % NSLOT), (seq - 1, (seq - 1) % NSLOT)):
    drain_gather(slot)
    wait_out(s - NSLOT, slot)
    transpose_into(slot)
    fire_out(s, slot)
  drain_gather((seq - 1) % NSLOT + 1)
  wait_out(seq - 3, (seq - 3) % NSLOT)
  wait_out(seq - 2, (seq - 2) % NSLOT)
  wait_out(seq - 1, (seq - 1) % NSLOT)


PREP_CB = 32768


def _prep_body(tn_ref, out_ref):
  hb = PREP_CB // 2
  xt = jnp.transpose(tn_ref[...])
  out_ref[...] = jnp.concatenate([xt[:hb], xt[hb:]], axis=1)


def _prep_table(table):
  n_rows, d = table.shape
  cb = PREP_CB
  n_blk = (n_rows + cb - 1) // cb
  tn = jnp.transpose(table)
  return pl.pallas_call(
      _prep_body,
      grid=(n_blk,),
      in_specs=[pl.BlockSpec((d, cb), lambda i: (0, i))],
      out_specs=pl.BlockSpec((cb // 2, 2 * d), lambda i: (i, 0)),
      out_shape=jax.ShapeDtypeStruct((n_blk * (cb // 2), 2 * d),
                                     jnp.float32),
  )(tn)


@jax.jit
def kernel(input_ids, table):
  batch, seq = input_ids.shape
  n_rows, d = table.shape
  assert batch % (NW * BLK) == 0 or batch == NW * BLK

  ids_t = jnp.transpose(input_ids)
  tbl2 = _prep_table(table)

  mesh = plsc.VectorSubcoreMesh(core_axis_name="c", subcore_axis_name="s")
  run = pl.kernel(
      functools.partial(_emb_body, seq, d),
      out_type=jax.ShapeDtypeStruct((seq, d, batch), jnp.float32),
      mesh=mesh,
      compiler_params=pltpu.CompilerParams(use_tc_tiling_on_sc=True,
                                           needs_layout_passes=False),
      scratch_types=(
          [pltpu.VMEM((seq, BLK), jnp.int32),
           pltpu.VMEM((NSLOT, 2, BLK), jnp.int32)]
          + [pltpu.VMEM((BLK, 2 * d), jnp.float32)] * NSLOT
          + [pltpu.VMEM((d, BLK), jnp.float32)] * NSLOT
          + [pltpu.SemaphoreType.DMA] * (2 * NSLOT)
      ),
  )
  out_t = run(ids_t, tbl2)
  return jnp.transpose(out_t, (2, 0, 1))
